# baseline (speedup 1.0000x reference)
import functools

import jax
import jax.numpy as jnp
from jax import lax
from jax.experimental import pallas as pl
from jax.experimental.pallas import tpu as pltpu
from jax.experimental.pallas import tpu_sc as plsc

N_NODE = 10000
E_TOT = 320000
H = 128

NR = 64
RNODE = 160
MAGIC = 26215
RSHIFT = 22
RBLK = 256
NRB = E_TOT // RBLK
SORT_CAP = 323584


def _stage_a_body(nf_gc, nf_gs, wgc, bgc, wgs, bgs, w1s2c, b1s2c, w1c2s, b1c2s,
                  gc_x, gs_x, ps_s2c, pd_s2c, ps_c2s, pd_c2s):
    xc = nf_gc[...] @ wgc[...] + bgc[...]
    xs = nf_gs[...] @ wgs[...] + bgs[...]
    gc_x[...] = xc
    gs_x[...] = xs
    ps_s2c[...] = xs @ w1s2c[0:H, :]
    pd_s2c[...] = xc @ w1s2c[H:2 * H, :] + b1s2c[...]
    ps_c2s[...] = xc @ w1c2s[0:H, :]
    pd_c2s[...] = xs @ w1c2s[H:2 * H, :] + b1c2s[...]


def _stage_a(nf_gc, nf_gs, wgc, bgc, wgs, bgs, w1s2c, b1s2c, w1c2s, b1c2s):
    n = nf_gc.shape[0]
    blk = 1000
    grid = n // blk
    row_spec = pl.BlockSpec((blk, H), lambda i: (i, 0))
    out_spec = pl.BlockSpec((blk, 2 * H), lambda i: (i, 0))
    full = lambda shape: pl.BlockSpec(shape, lambda i: tuple(0 for _ in shape))
    return pl.pallas_call(
        _stage_a_body,
        grid=(grid,),
        in_specs=[row_spec, row_spec,
                  full((H, H)), full((1, H)), full((H, H)), full((1, H)),
                  full((2 * H, 2 * H)), full((1, 2 * H)),
                  full((2 * H, 2 * H)), full((1, 2 * H))],
        out_specs=(row_spec, row_spec, out_spec, out_spec, out_spec, out_spec),
        out_shape=(
            jax.ShapeDtypeStruct((n, H), jnp.float32),
            jax.ShapeDtypeStruct((n, H), jnp.float32),
            jax.ShapeDtypeStruct((n, 2 * H), jnp.float32),
            jax.ShapeDtypeStruct((n, 2 * H), jnp.float32),
            jax.ShapeDtypeStruct((n, 2 * H), jnp.float32),
            jax.ShapeDtypeStruct((n, 2 * H), jnp.float32),
        ),
    )(nf_gc, nf_gs, wgc, bgc.reshape(1, -1), wgs, bgs.reshape(1, -1),
      w1s2c, b1s2c.reshape(1, -1), w1c2s, b1c2s.reshape(1, -1))


def _prep_body(w2_1, w2_4, wr1, wr4, b2_1, b2_4, wm, bm):
    wm[...] = w2_1[...] @ wr1[...] + w2_4[...] @ wr4[...]
    bm[...] = b2_1[...] @ wr1[...] + b2_4[...] @ wr4[...]


def _prep(w2, b2, red_w):
    wm, bm = pl.pallas_call(
        _prep_body,
        out_shape=(jax.ShapeDtypeStruct((2 * H, H), jnp.float32),
                   jax.ShapeDtypeStruct((1, H), jnp.float32)),
    )(w2[:, 1:1 + H], w2[:, 1 + 3 * H:1 + 4 * H],
      red_w[H:2 * H], red_w[4 * H:5 * H],
      b2[1:1 + H].reshape(1, H), b2[1 + 3 * H:1 + 4 * H].reshape(1, H))
    aux_w = jnp.pad(w2[:, 0:1], ((0, 0), (0, H - 1)))
    aux_b = jnp.pad(b2[0:1], (0, H - 1)).reshape(1, H)
    w2cat = jnp.concatenate(
        [wm, w2[:, 1 + H:1 + 2 * H], w2[:, 1 + 2 * H:1 + 3 * H], aux_w], axis=1)
    b2cat = jnp.concatenate(
        [bm, b2[1 + H:1 + 2 * H].reshape(1, H),
         b2[1 + 2 * H:1 + 3 * H].reshape(1, H), aux_b], axis=1)
    return w2cat, b2cat


def _count_body(dst2d, bc):
    r = lax.shift_right_logical(dst2d[...] * MAGIC, RSHIFT)
    oh = (r == lax.broadcasted_iota(jnp.int32, (RBLK, NR), 1))
    bc[...] = jnp.sum(oh.astype(jnp.float32), axis=0,
                      keepdims=True).astype(jnp.int32).reshape(1, 1, NR)


def _count(dst2d):
    return pl.pallas_call(
        _count_body,
        grid=(NRB,),
        in_specs=[pl.BlockSpec((RBLK, 1), lambda i: (i, 0))],
        out_specs=pl.BlockSpec((1, 1, NR), lambda i: (i, 0, 0)),
        out_shape=jax.ShapeDtypeStruct((NRB, 1, NR), jnp.int32),
    )(dst2d)


def _prefix_body(bc, off, seg):
    c = bc[...].reshape(NRB, NR).astype(jnp.float32)
    ii = lax.broadcasted_iota(jnp.int32, (NRB, NRB), 0)
    jj = lax.broadcasted_iota(jnp.int32, (NRB, NRB), 1)
    tri = (jj < ii).astype(jnp.float32)
    p = tri @ c
    total = jnp.sum(c, axis=0, keepdims=True)
    pad_tot = (jnp.floor(total / 8.0 + 0.875)) * 8.0
    i2 = lax.broadcasted_iota(jnp.int32, (NR, NR), 0)
    j2 = lax.broadcasted_iota(jnp.int32, (NR, NR), 1)
    tri2 = (i2 < j2).astype(jnp.float32)
    offs = pad_tot @ tri2
    off[...] = (p + offs).astype(jnp.int32).reshape(NRB, 1, NR)
    seg[...] = jnp.concatenate(
        [offs, total, jnp.zeros((1, 32), jnp.float32)], axis=1).astype(jnp.int32)


def _prefix(bc):
    return pl.pallas_call(
        _prefix_body,
        out_shape=(jax.ShapeDtypeStruct((NRB, 1, NR), jnp.int32),
                   jax.ShapeDtypeStruct((1, NR + 96), jnp.int32)),
    )(bc)


def _pos_body(dst2d, off, pos):
    r = lax.shift_right_logical(dst2d[...] * MAGIC, RSHIFT)
    oh = (r == lax.broadcasted_iota(jnp.int32, (RBLK, NR), 1)).astype(jnp.float32)
    ii = lax.broadcasted_iota(jnp.int32, (RBLK, RBLK), 0)
    jj = lax.broadcasted_iota(jnp.int32, (RBLK, RBLK), 1)
    tri = (jj < ii).astype(jnp.float32)
    rank = jnp.sum((tri @ oh) * oh, axis=1, keepdims=True)
    offr = off[...].reshape(1, NR).astype(jnp.float32)
    base = jnp.sum(offr * oh, axis=1, keepdims=True)
    pos[...] = (rank + base).astype(jnp.int32)


def _pos(dst2d, off):
    return pl.pallas_call(
        _pos_body,
        grid=(NRB,),
        in_specs=[pl.BlockSpec((RBLK, 1), lambda i: (i, 0)),
                  pl.BlockSpec((1, 1, NR), lambda i: (i, 0, 0))],
        out_specs=pl.BlockSpec((RBLK, 1), lambda i: (i, 0)),
        out_shape=jax.ShapeDtypeStruct((E_TOT, 1), jnp.int32),
    )(dst2d, off)


_SC_E_W = E_TOT // 32
_SC_CH = 80
_SC_NCH = _SC_E_W // _SC_CH


def _scatter_body(src, dst, pos, ss, dd, sbuf, dbuf, pbuf, sem1, sem2):
    info = plsc.get_sparse_core_info()
    nc = info.num_cores
    wid = lax.axis_index("s") * nc + lax.axis_index("c")
    base = wid * _SC_E_W

    def step(c, carry):
        off = base + c * _SC_CH
        pltpu.sync_copy(src.at[pl.ds(off, _SC_CH)], sbuf)
        pltpu.sync_copy(dst.at[pl.ds(off, _SC_CH)], dbuf)
        pltpu.sync_copy(pos.at[pl.ds(off, _SC_CH)], pbuf)
        cp1 = pltpu.async_copy(sbuf, ss.at[pbuf], sem1)
        cp2 = pltpu.async_copy(dbuf, dd.at[pbuf], sem2)
        cp1.wait()
        cp2.wait()
        return carry

    lax.fori_loop(0, _SC_NCH, step, 0)


def _sc_scatter(src, dst, pos):
    mesh = plsc.VectorSubcoreMesh(core_axis_name="c", subcore_axis_name="s")
    f = pl.kernel(
        _scatter_body,
        out_type=(jax.ShapeDtypeStruct((SORT_CAP,), jnp.int32),
                  jax.ShapeDtypeStruct((SORT_CAP,), jnp.int32)),
        mesh=mesh,
        scratch_types=[
            pltpu.VMEM((_SC_CH,), jnp.int32),
            pltpu.VMEM((_SC_CH,), jnp.int32),
            pltpu.VMEM((_SC_CH,), jnp.int32),
            pltpu.SemaphoreType.DMA,
            pltpu.SemaphoreType.DMA,
        ],
    )
    return f(src, dst, pos)


_G_W = SORT_CAP // 32
_G_CH = 128
_G_NCH = _G_W // _G_CH


def _gather_body(ps, pd, src, dst, ea, eb, idxs, idxd, bufa, bufb, sem1, sem2):
    info = plsc.get_sparse_core_info()
    nc = info.num_cores
    wid = lax.axis_index("s") * nc + lax.axis_index("c")
    base = wid * _G_W

    def step(c, carry):
        off = base + c * _G_CH
        pltpu.sync_copy(src.at[pl.ds(off, _G_CH)], idxs)
        pltpu.sync_copy(dst.at[pl.ds(off, _G_CH)], idxd)
        for j in range(_G_CH // 16):
            sl = pl.ds(j * 16, 16)
            idxs[sl] = jnp.clip(idxs[sl], 0, N_NODE - 1)
            idxd[sl] = jnp.clip(idxd[sl], 0, N_NODE - 1)
        cp1 = pltpu.async_copy(ps.at[idxs], bufa, sem1)
        cp2 = pltpu.async_copy(pd.at[idxd], bufb, sem2)
        cp1.wait()
        cp2.wait()
        pltpu.sync_copy(bufa, ea.at[pl.ds(off, _G_CH)])
        pltpu.sync_copy(bufb, eb.at[pl.ds(off, _G_CH)])
        return carry

    lax.fori_loop(0, _G_NCH, step, 0)


def _sc_gather(ps, pd, src, dst):
    mesh = plsc.VectorSubcoreMesh(core_axis_name="c", subcore_axis_name="s")
    f = pl.kernel(
        _gather_body,
        out_type=(jax.ShapeDtypeStruct((SORT_CAP, 2 * H), jnp.float32),
                  jax.ShapeDtypeStruct((SORT_CAP, 2 * H), jnp.float32)),
        mesh=mesh,
        scratch_types=[
            pltpu.VMEM((_G_CH,), jnp.int32),
            pltpu.VMEM((_G_CH,), jnp.int32),
            pltpu.VMEM((_G_CH, 2 * H), jnp.float32),
            pltpu.VMEM((_G_CH, 2 * H), jnp.float32),
            pltpu.SemaphoreType.DMA,
            pltpu.SemaphoreType.DMA,
        ],
    )
    return f(ps, pd, src, dst)


def _stage_c_body(ea, eb, w2cat, b2cat, g):
    z = jax.nn.relu(ea[...] + eb[...])
    h = z @ w2cat[...] + b2cat[...]
    k = jax.nn.sigmoid(h[:, 3 * H:3 * H + 1])
    g[...] = h[:, 0:3 * H] * k


def _stage_c(ea, eb, w2cat, b2cat):
    blk = 512
    grid = SORT_CAP // blk
    espec = pl.BlockSpec((blk, 2 * H), lambda i: (i, 0))
    return pl.pallas_call(
        _stage_c_body,
        grid=(grid,),
        in_specs=[espec, espec,
                  pl.BlockSpec((2 * H, 4 * H), lambda i: (0, 0)),
                  pl.BlockSpec((1, 4 * H), lambda i: (0, 0))],
        out_specs=pl.BlockSpec((blk, 3 * H), lambda i: (i, 0)),
        out_shape=jax.ShapeDtypeStruct((SORT_CAP, 3 * H), jnp.float32),
    )(ea, eb, w2cat, b2cat)


_D_CH = 64


def _reduce_body(g, dsts, seg, out, sibuf, dbuf, gbuf, acc, sem):
    info = plsc.get_sparse_core_info()
    nc = info.num_cores
    wid = lax.axis_index("s") * nc + lax.axis_index("c")
    pltpu.sync_copy(seg.at[pl.ds(wid * 16, 16)], sibuf)
    iota = lax.broadcasted_iota(jnp.int32, (16,), 0)
    sv = sibuf[pl.ds(0, 16)]

    def range_pass(p, carry0):
        rid = wid + 32 * p
        lo = rid * RNODE
        start = pl.multiple_of(sv[2 * p], 8)
        n = sv[2 * p + 1]

        def init_acc(rr, c):
            for j in range(8):
                acc[rr, pl.ds(j * 16, 16)] = jnp.zeros((16,), jnp.float32)
            for j in range(8, 16):
                acc[rr, pl.ds(j * 16, 16)] = jnp.full((16,), -jnp.inf,
                                                      jnp.float32)
            for j in range(16, 24):
                acc[rr, pl.ds(j * 16, 16)] = jnp.full((16,), jnp.inf,
                                                      jnp.float32)
            return c
        lax.fori_loop(0, RNODE + 8, init_acc, 0)

        def chunk(cc, carry1):
            @pl.when(cc * _D_CH < n)
            def _():
                off = start + cc * _D_CH
                pltpu.sync_copy(dsts.at[pl.ds(off, _D_CH)], dbuf)
                cp = pltpu.async_copy(g.at[pl.ds(off, _D_CH)], gbuf, sem)
                cp.wait()
                nb = n - cc * _D_CH

                def group(gg, carry2):
                    dv = dbuf[pl.ds(gg * 16, 16)]
                    valid = (gg * 16 + iota) < nb
                    dlv = jnp.where(valid, dv - lo, RNODE)
                    for lane in range(16):
                        dl = dlv[lane]
                        rl = gg * 16 + lane
                        for j in range(8):
                            cs = pl.ds(j * 16, 16)
                            acc[dl, cs] = acc[dl, cs] + gbuf[rl, cs]
                        for j in range(8, 16):
                            cs = pl.ds(j * 16, 16)
                            acc[dl, cs] = jnp.maximum(acc[dl, cs],
                                                      gbuf[rl, cs])
                        for j in range(16, 24):
                            cs = pl.ds(j * 16, 16)
                            acc[dl, cs] = jnp.minimum(acc[dl, cs],
                                                      gbuf[rl, cs])
                    return carry2

                lax.fori_loop(0, _D_CH // 16, group, 0)
            return carry1

        lax.fori_loop(0, SORT_CAP // _D_CH, chunk, 0)
        pltpu.sync_copy(acc.at[pl.ds(0, RNODE)], out.at[pl.ds(lo, RNODE)])
        return carry0

    range_pass(0, 0)
    range_pass(1, 0)


def _sc_reduce(g, dsts, seg):
    mesh = plsc.VectorSubcoreMesh(core_axis_name="c", subcore_axis_name="s")
    f = pl.kernel(
        _reduce_body,
        out_type=jax.ShapeDtypeStruct((NR * RNODE, 3 * H), jnp.float32),
        mesh=mesh,
        scratch_types=[
            pltpu.VMEM((16,), jnp.int32),
            pltpu.VMEM((_D_CH,), jnp.int32),
            pltpu.VMEM((_D_CH, 3 * H), jnp.float32),
            pltpu.VMEM((RNODE + 8, 3 * H), jnp.float32),
            pltpu.SemaphoreType.DMA,
        ],
    )
    return f(g, dsts, seg)


def _stage_e_body(gcx, msc, a2c, a3c, gsx, mss, a2s, a3s,
                  rwc0, rwc2, rwc3, rbc, rws0, rws2, rws3, rbs,
                  cw1a, cw1b, cb1, cw2, cb2, sw1a, sw1b, sb1, sw2, sb2,
                  out_fc, out_fs):
    new_cx = gcx[...] @ rwc0[...] + msc[...] + a2c[...] @ rwc2[...] \
        + a3c[...] @ rwc3[...] + rbc[...]
    t = jax.nn.relu(gcx[...] @ cw1a[...] + new_cx @ cw1b[...] + cb1[...])
    out_fc[...] = t @ cw2[...] + cb2[...]
    new_sx = gsx[...] @ rws0[...] + mss[...] + a2s[...] @ rws2[...] \
        + a3s[...] @ rws3[...] + rbs[...]
    u = jax.nn.relu(gsx[...] @ sw1a[...] + new_sx @ sw1b[...] + sb1[...])
    out_fs[...] = u @ sw2[...] + sb2[...]


def _stage_e(gc_x, msc, a2c, a3c, gs_x, mss, a2s, a3s,
             red_s2c_w, red_s2c_b, red_c2s_w, red_c2s_b,
             gc_w1, gc_b1, gc_w2, gc_b2, gs_w1, gs_b1, gs_w2, gs_b2):
    n = gc_x.shape[0]
    blk = 1000
    grid = n // blk
    row = pl.BlockSpec((blk, H), lambda i: (i, 0))
    wfull = pl.BlockSpec((H, H), lambda i: (0, 0))
    bfull = pl.BlockSpec((1, H), lambda i: (0, 0))
    return pl.pallas_call(
        _stage_e_body,
        grid=(grid,),
        in_specs=[row] * 8 + [wfull, wfull, wfull, bfull] * 2
        + [wfull, wfull, bfull, wfull, bfull] * 2,
        out_specs=(row, row),
        out_shape=(jax.ShapeDtypeStruct((n, H), jnp.float32),
                   jax.ShapeDtypeStruct((n, H), jnp.float32)),
    )(gc_x, msc, a2c, a3c, gs_x, mss, a2s, a3s,
      red_s2c_w[0:H], red_s2c_w[2 * H:3 * H], red_s2c_w[3 * H:4 * H],
      red_s2c_b.reshape(1, H),
      red_c2s_w[0:H], red_c2s_w[2 * H:3 * H], red_c2s_w[3 * H:4 * H],
      red_c2s_b.reshape(1, H),
      gc_w1[0:H], gc_w1[H:2 * H], gc_b1.reshape(1, H), gc_w2,
      gc_b2.reshape(1, H),
      gs_w1[0:H], gs_w1[H:2 * H], gs_b1.reshape(1, H), gs_w2,
      gs_b2.reshape(1, H))


def _edge_reduce(ps, pd, ei, w2cat, b2cat):
    src, dst = ei[0], ei[1]
    bc = _count(dst.reshape(E_TOT, 1))
    off, seg = _prefix(bc)
    pos = _pos(dst.reshape(E_TOT, 1), off).reshape(-1)
    ss, dd = _sc_scatter(src, dst, pos)
    ea, eb = _sc_gather(ps, pd, ss, dd)
    g = _stage_c(ea, eb, w2cat, b2cat)
    vmask = jnp.zeros((SORT_CAP,), bool).at[pos].set(True)
    sid = jnp.where(vmask, dd, N_NODE * 2)
    ms = jax.ops.segment_sum(g[:, 0:H], sid, num_segments=NR * RNODE)
    a2 = jax.ops.segment_max(g[:, H:2 * H], sid, num_segments=NR * RNODE)
    a3 = jax.ops.segment_min(g[:, 2 * H:3 * H], sid, num_segments=NR * RNODE)
    return jnp.concatenate([ms, a2, a3], axis=1)


def kernel(nf_gc, nf_gs, ei_s2c, ei_c2s, lin_gc_w, lin_gc_b, lin_gs_w, lin_gs_b,
           msg_s2c_w1, msg_s2c_b1, msg_s2c_w2, msg_s2c_b2,
           red_s2c_w, red_s2c_b,
           msg_c2s_w1, msg_c2s_b1, msg_c2s_w2, msg_c2s_b2,
           red_c2s_w, red_c2s_b,
           gc_w1, gc_b1, gc_w2, gc_b2,
           gs_w1, gs_b1, gs_w2, gs_b2):
    gc_x, gs_x, ps_s2c, pd_s2c, ps_c2s, pd_c2s = _stage_a(
        nf_gc, nf_gs, lin_gc_w, lin_gc_b, lin_gs_w, lin_gs_b,
        msg_s2c_w1, msg_s2c_b1, msg_c2s_w1, msg_c2s_b1)
    w2cat_c, b2cat_c = _prep(msg_s2c_w2, msg_s2c_b2, red_s2c_w)
    w2cat_s, b2cat_s = _prep(msg_c2s_w2, msg_c2s_b2, red_c2s_w)

    red_c = _edge_reduce(ps_s2c, pd_s2c, ei_s2c, w2cat_c, b2cat_c)
    red_s = _edge_reduce(ps_c2s, pd_c2s, ei_c2s, w2cat_s, b2cat_s)

    msc = red_c[:N_NODE, 0:H]
    a2c = red_c[:N_NODE, H:2 * H]
    a3c = red_c[:N_NODE, 2 * H:3 * H]
    mss = red_s[:N_NODE, 0:H]
    a2s = red_s[:N_NODE, H:2 * H]
    a3s = red_s[:N_NODE, 2 * H:3 * H]

    return _stage_e(gc_x, msc, a2c, a3c, gs_x, mss, a2s, a3s,
                    red_s2c_w, red_s2c_b, red_c2s_w, red_c2s_b,
                    gc_w1, gc_b1, gc_w2, gc_b2, gs_w1, gs_b1, gs_w2, gs_b2)


# speedup vs baseline: 2.0918x; 2.0918x over previous
import functools

import jax
import jax.numpy as jnp
from jax import lax
from jax.experimental import pallas as pl
from jax.experimental.pallas import tpu as pltpu
from jax.experimental.pallas import tpu_sc as plsc

N_NODE = 10000
E_TOT = 320000
H = 128


def _stage_a_body(nf_gc, nf_gs, wgc, bgc, wgs, bgs, w1s2c, b1s2c, w1c2s, b1c2s,
                  gc_x, gs_x, ps_s2c, pd_s2c, ps_c2s, pd_c2s):
    xc = nf_gc[...] @ wgc[...] + bgc[...]
    xs = nf_gs[...] @ wgs[...] + bgs[...]
    gc_x[...] = xc
    gs_x[...] = xs
    ps_s2c[...] = xs @ w1s2c[0:H, :]
    pd_s2c[...] = xc @ w1s2c[H:2 * H, :] + b1s2c[...]
    ps_c2s[...] = xc @ w1c2s[0:H, :]
    pd_c2s[...] = xs @ w1c2s[H:2 * H, :] + b1c2s[...]


def _stage_a(nf_gc, nf_gs, wgc, bgc, wgs, bgs, w1s2c, b1s2c, w1c2s, b1c2s):
    n = nf_gc.shape[0]
    blk = 1000
    grid = n // blk
    row_spec = pl.BlockSpec((blk, H), lambda i: (i, 0))
    out_spec = pl.BlockSpec((blk, 2 * H), lambda i: (i, 0))
    full = lambda shape: pl.BlockSpec(shape, lambda i: tuple(0 for _ in shape))
    return pl.pallas_call(
        _stage_a_body,
        grid=(grid,),
        in_specs=[row_spec, row_spec,
                  full((H, H)), full((1, H)), full((H, H)), full((1, H)),
                  full((2 * H, 2 * H)), full((1, 2 * H)),
                  full((2 * H, 2 * H)), full((1, 2 * H))],
        out_specs=(row_spec, row_spec, out_spec, out_spec, out_spec, out_spec),
        out_shape=(
            jax.ShapeDtypeStruct((n, H), jnp.float32),
            jax.ShapeDtypeStruct((n, H), jnp.float32),
            jax.ShapeDtypeStruct((n, 2 * H), jnp.float32),
            jax.ShapeDtypeStruct((n, 2 * H), jnp.float32),
            jax.ShapeDtypeStruct((n, 2 * H), jnp.float32),
            jax.ShapeDtypeStruct((n, 2 * H), jnp.float32),
        ),
    )(nf_gc, nf_gs, wgc, bgc.reshape(1, -1), wgs, bgs.reshape(1, -1),
      w1s2c, b1s2c.reshape(1, -1), w1c2s, b1c2s.reshape(1, -1))


def _prep_body(w2_1, w2_4, wr1, wr4, b2_1, b2_4, wm, bm):
    wm[...] = w2_1[...] @ wr1[...] + w2_4[...] @ wr4[...]
    bm[...] = b2_1[...] @ wr1[...] + b2_4[...] @ wr4[...]


def _prep(w2, b2, red_w):
    wm, bm = pl.pallas_call(
        _prep_body,
        out_shape=(jax.ShapeDtypeStruct((2 * H, H), jnp.float32),
                   jax.ShapeDtypeStruct((1, H), jnp.float32)),
    )(w2[:, 1:1 + H], w2[:, 1 + 3 * H:1 + 4 * H],
      red_w[H:2 * H], red_w[4 * H:5 * H],
      b2[1:1 + H].reshape(1, H), b2[1 + 3 * H:1 + 4 * H].reshape(1, H))
    aux_w = jnp.pad(w2[:, 0:1], ((0, 0), (0, H - 1)))
    aux_b = jnp.pad(b2[0:1], (0, H - 1)).reshape(1, H)
    w2cat = jnp.concatenate(
        [wm, w2[:, 1 + H:1 + 2 * H], w2[:, 1 + 2 * H:1 + 3 * H], aux_w], axis=1)
    b2cat = jnp.concatenate(
        [bm, b2[1 + H:1 + 2 * H].reshape(1, H),
         b2[1 + 2 * H:1 + 3 * H].reshape(1, H), aux_b], axis=1)
    return w2cat, b2cat


_E_PER_W = E_TOT // 32
_CH = 80
_NCH = _E_PER_W // _CH


def _gather_body(ps, pd, src, dst, ea, eb, idxs, idxd, bufa, bufb, sem1, sem2):
    info = plsc.get_sparse_core_info()
    nc = info.num_cores
    wid = lax.axis_index("s") * nc + lax.axis_index("c")
    base = wid * _E_PER_W

    def step(c, carry):
        off = base + c * _CH
        pltpu.sync_copy(src.at[pl.ds(off, _CH)], idxs)
        pltpu.sync_copy(dst.at[pl.ds(off, _CH)], idxd)
        cp1 = pltpu.async_copy(ps.at[idxs], bufa, sem1)
        cp2 = pltpu.async_copy(pd.at[idxd], bufb, sem2)
        cp1.wait()
        cp2.wait()
        pltpu.sync_copy(bufa, ea.at[pl.ds(off, _CH)])
        pltpu.sync_copy(bufb, eb.at[pl.ds(off, _CH)])
        return carry

    lax.fori_loop(0, _NCH, step, 0)


def _sc_gather(ps, pd, src, dst):
    mesh = plsc.VectorSubcoreMesh(core_axis_name="c", subcore_axis_name="s")
    f = pl.kernel(
        _gather_body,
        out_type=(jax.ShapeDtypeStruct((E_TOT, 2 * H), jnp.float32),
                  jax.ShapeDtypeStruct((E_TOT, 2 * H), jnp.float32)),
        mesh=mesh,
        scratch_types=[
            pltpu.VMEM((_CH,), jnp.int32),
            pltpu.VMEM((_CH,), jnp.int32),
            pltpu.VMEM((_CH, 2 * H), jnp.float32),
            pltpu.VMEM((_CH, 2 * H), jnp.float32),
            pltpu.SemaphoreType.DMA,
            pltpu.SemaphoreType.DMA,
        ],
    )
    return f(ps, pd, src, dst)


def _stage_c_body(ea, eb, w2cat, b2cat, g):
    z = jax.nn.relu(ea[...] + eb[...])
    h = z @ w2cat[...] + b2cat[...]
    k = jax.nn.sigmoid(h[:, 3 * H:3 * H + 1])
    g[...] = h[:, 0:3 * H] * k


def _stage_c(ea, eb, w2cat, b2cat):
    blk = 512
    grid = E_TOT // blk
    espec = pl.BlockSpec((blk, 2 * H), lambda i: (i, 0))
    return pl.pallas_call(
        _stage_c_body,
        grid=(grid,),
        in_specs=[espec, espec,
                  pl.BlockSpec((2 * H, 4 * H), lambda i: (0, 0)),
                  pl.BlockSpec((1, 4 * H), lambda i: (0, 0))],
        out_specs=pl.BlockSpec((blk, 3 * H), lambda i: (i, 0)),
        out_shape=jax.ShapeDtypeStruct((E_TOT, 3 * H), jnp.float32),
    )(ea, eb, w2cat, b2cat)


def _stage_e_body(gcx, msc, a2c, a3c, gsx, mss, a2s, a3s,
                  rwc0, rwc2, rwc3, rbc, rws0, rws2, rws3, rbs,
                  cw1a, cw1b, cb1, cw2, cb2, sw1a, sw1b, sb1, sw2, sb2,
                  out_fc, out_fs):
    new_cx = gcx[...] @ rwc0[...] + msc[...] + a2c[...] @ rwc2[...] \
        + a3c[...] @ rwc3[...] + rbc[...]
    t = jax.nn.relu(gcx[...] @ cw1a[...] + new_cx @ cw1b[...] + cb1[...])
    out_fc[...] = t @ cw2[...] + cb2[...]
    new_sx = gsx[...] @ rws0[...] + mss[...] + a2s[...] @ rws2[...] \
        + a3s[...] @ rws3[...] + rbs[...]
    u = jax.nn.relu(gsx[...] @ sw1a[...] + new_sx @ sw1b[...] + sb1[...])
    out_fs[...] = u @ sw2[...] + sb2[...]


def _stage_e(gc_x, msc, a2c, a3c, gs_x, mss, a2s, a3s,
             red_s2c_w, red_s2c_b, red_c2s_w, red_c2s_b,
             gc_w1, gc_b1, gc_w2, gc_b2, gs_w1, gs_b1, gs_w2, gs_b2):
    n = gc_x.shape[0]
    blk = 1000
    grid = n // blk
    row = pl.BlockSpec((blk, H), lambda i: (i, 0))
    wfull = pl.BlockSpec((H, H), lambda i: (0, 0))
    bfull = pl.BlockSpec((1, H), lambda i: (0, 0))
    return pl.pallas_call(
        _stage_e_body,
        grid=(grid,),
        in_specs=[row] * 8 + [wfull, wfull, wfull, bfull] * 2
        + [wfull, wfull, bfull, wfull, bfull] * 2,
        out_specs=(row, row),
        out_shape=(jax.ShapeDtypeStruct((n, H), jnp.float32),
                   jax.ShapeDtypeStruct((n, H), jnp.float32)),
    )(gc_x, msc, a2c, a3c, gs_x, mss, a2s, a3s,
      red_s2c_w[0:H], red_s2c_w[2 * H:3 * H], red_s2c_w[3 * H:4 * H],
      red_s2c_b.reshape(1, H),
      red_c2s_w[0:H], red_c2s_w[2 * H:3 * H], red_c2s_w[3 * H:4 * H],
      red_c2s_b.reshape(1, H),
      gc_w1[0:H], gc_w1[H:2 * H], gc_b1.reshape(1, H), gc_w2,
      gc_b2.reshape(1, H),
      gs_w1[0:H], gs_w1[H:2 * H], gs_b1.reshape(1, H), gs_w2,
      gs_b2.reshape(1, H))


def _edge_payload(ps, pd, src, dst, w2cat, b2cat):
    ea, eb = _sc_gather(ps, pd, src, dst)
    return _stage_c(ea, eb, w2cat, b2cat)


def kernel(nf_gc, nf_gs, ei_s2c, ei_c2s, lin_gc_w, lin_gc_b, lin_gs_w, lin_gs_b,
           msg_s2c_w1, msg_s2c_b1, msg_s2c_w2, msg_s2c_b2,
           red_s2c_w, red_s2c_b,
           msg_c2s_w1, msg_c2s_b1, msg_c2s_w2, msg_c2s_b2,
           red_c2s_w, red_c2s_b,
           gc_w1, gc_b1, gc_w2, gc_b2,
           gs_w1, gs_b1, gs_w2, gs_b2):
    gc_x, gs_x, ps_s2c, pd_s2c, ps_c2s, pd_c2s = _stage_a(
        nf_gc, nf_gs, lin_gc_w, lin_gc_b, lin_gs_w, lin_gs_b,
        msg_s2c_w1, msg_s2c_b1, msg_c2s_w1, msg_c2s_b1)
    w2cat_c, b2cat_c = _prep(msg_s2c_w2, msg_s2c_b2, red_s2c_w)
    w2cat_s, b2cat_s = _prep(msg_c2s_w2, msg_c2s_b2, red_c2s_w)

    g_c = _edge_payload(ps_s2c, pd_s2c, ei_s2c[0], ei_s2c[1], w2cat_c, b2cat_c)
    g_s = _edge_payload(ps_c2s, pd_c2s, ei_c2s[0], ei_c2s[1], w2cat_s, b2cat_s)

    dst_c = ei_s2c[1]
    dst_s = ei_c2s[1]
    msc = jax.ops.segment_sum(g_c[:, 0:H], dst_c, num_segments=N_NODE)
    a2c = jax.ops.segment_max(g_c[:, H:2 * H], dst_c, num_segments=N_NODE)
    a3c = jax.ops.segment_min(g_c[:, 2 * H:3 * H], dst_c, num_segments=N_NODE)
    mss = jax.ops.segment_sum(g_s[:, 0:H], dst_s, num_segments=N_NODE)
    a2s = jax.ops.segment_max(g_s[:, H:2 * H], dst_s, num_segments=N_NODE)
    a3s = jax.ops.segment_min(g_s[:, 2 * H:3 * H], dst_s, num_segments=N_NODE)

    return _stage_e(gc_x, msc, a2c, a3c, gs_x, mss, a2s, a3s,
                    red_s2c_w, red_s2c_b, red_c2s_w, red_c2s_b,
                    gc_w1, gc_b1, gc_w2, gc_b2, gs_w1, gs_b1, gs_w2, gs_b2)


# speedup vs baseline: 2.1156x; 1.0114x over previous
import functools

import jax
import jax.numpy as jnp
from jax import lax
from jax.experimental import pallas as pl
from jax.experimental.pallas import tpu as pltpu
from jax.experimental.pallas import tpu_sc as plsc

N_NODE = 10000
E_TOT = 320000
H = 128


def _stage_a_body(nf_gc, nf_gs, wgc, bgc, wgs, bgs, w1s2c, b1s2c, w1c2s, b1c2s,
                  gc_x, gs_x, ps_s2c, pd_s2c, ps_c2s, pd_c2s):
    xc = nf_gc[...] @ wgc[...] + bgc[...]
    xs = nf_gs[...] @ wgs[...] + bgs[...]
    gc_x[...] = xc
    gs_x[...] = xs
    ps_s2c[...] = (xs @ w1s2c[0:H, :]).astype(jnp.bfloat16)
    pd_s2c[...] = (xc @ w1s2c[H:2 * H, :] + b1s2c[...]).astype(jnp.bfloat16)
    ps_c2s[...] = (xc @ w1c2s[0:H, :]).astype(jnp.bfloat16)
    pd_c2s[...] = (xs @ w1c2s[H:2 * H, :] + b1c2s[...]).astype(jnp.bfloat16)


def _stage_a(nf_gc, nf_gs, wgc, bgc, wgs, bgs, w1s2c, b1s2c, w1c2s, b1c2s):
    n = nf_gc.shape[0]
    blk = 1000
    grid = n // blk
    row_spec = pl.BlockSpec((blk, H), lambda i: (i, 0))
    out_spec = pl.BlockSpec((blk, 2 * H), lambda i: (i, 0))
    full = lambda shape: pl.BlockSpec(shape, lambda i: tuple(0 for _ in shape))
    return pl.pallas_call(
        _stage_a_body,
        grid=(grid,),
        in_specs=[row_spec, row_spec,
                  full((H, H)), full((1, H)), full((H, H)), full((1, H)),
                  full((2 * H, 2 * H)), full((1, 2 * H)),
                  full((2 * H, 2 * H)), full((1, 2 * H))],
        out_specs=(row_spec, row_spec, out_spec, out_spec, out_spec, out_spec),
        out_shape=(
            jax.ShapeDtypeStruct((n, H), jnp.float32),
            jax.ShapeDtypeStruct((n, H), jnp.float32),
            jax.ShapeDtypeStruct((n, 2 * H), jnp.bfloat16),
            jax.ShapeDtypeStruct((n, 2 * H), jnp.bfloat16),
            jax.ShapeDtypeStruct((n, 2 * H), jnp.bfloat16),
            jax.ShapeDtypeStruct((n, 2 * H), jnp.bfloat16),
        ),
    )(nf_gc, nf_gs, wgc, bgc.reshape(1, -1), wgs, bgs.reshape(1, -1),
      w1s2c, b1s2c.reshape(1, -1), w1c2s, b1c2s.reshape(1, -1))


def _prep_body(w2_1, w2_4, wr1, wr4, b2_1, b2_4, wm, bm):
    wm[...] = w2_1[...] @ wr1[...] + w2_4[...] @ wr4[...]
    bm[...] = b2_1[...] @ wr1[...] + b2_4[...] @ wr4[...]


def _prep(w2, b2, red_w):
    wm, bm = pl.pallas_call(
        _prep_body,
        out_shape=(jax.ShapeDtypeStruct((2 * H, H), jnp.float32),
                   jax.ShapeDtypeStruct((1, H), jnp.float32)),
    )(w2[:, 1:1 + H], w2[:, 1 + 3 * H:1 + 4 * H],
      red_w[H:2 * H], red_w[4 * H:5 * H],
      b2[1:1 + H].reshape(1, H), b2[1 + 3 * H:1 + 4 * H].reshape(1, H))
    aux_w = jnp.pad(w2[:, 0:1], ((0, 0), (0, H - 1)))
    aux_b = jnp.pad(b2[0:1], (0, H - 1)).reshape(1, H)
    w2cat = jnp.concatenate(
        [wm, w2[:, 1 + H:1 + 2 * H], w2[:, 1 + 2 * H:1 + 3 * H], aux_w], axis=1)
    b2cat = jnp.concatenate(
        [bm, b2[1 + H:1 + 2 * H].reshape(1, H),
         b2[1 + 2 * H:1 + 3 * H].reshape(1, H), aux_b], axis=1)
    return w2cat, b2cat


_E_PER_W = E_TOT // 32
_CH = 80
_NCH = _E_PER_W // _CH


def _gather_body(ps, pd, src, dst, ea, eb, idxs, idxd, bufa, bufb, sem1, sem2):
    info = plsc.get_sparse_core_info()
    nc = info.num_cores
    wid = lax.axis_index("s") * nc + lax.axis_index("c")
    base = wid * _E_PER_W

    def step(c, carry):
        off = base + c * _CH
        pltpu.sync_copy(src.at[pl.ds(off, _CH)], idxs)
        pltpu.sync_copy(dst.at[pl.ds(off, _CH)], idxd)
        cp1 = pltpu.async_copy(ps.at[idxs], bufa, sem1)
        cp2 = pltpu.async_copy(pd.at[idxd], bufb, sem2)
        cp1.wait()
        cp2.wait()
        pltpu.sync_copy(bufa, ea.at[pl.ds(off, _CH)])
        pltpu.sync_copy(bufb, eb.at[pl.ds(off, _CH)])
        return carry

    lax.fori_loop(0, _NCH, step, 0)


def _sc_gather(ps, pd, src, dst):
    mesh = plsc.VectorSubcoreMesh(core_axis_name="c", subcore_axis_name="s")
    f = pl.kernel(
        _gather_body,
        out_type=(jax.ShapeDtypeStruct((E_TOT, H), jnp.int32),
                  jax.ShapeDtypeStruct((E_TOT, H), jnp.int32)),
        mesh=mesh,
        scratch_types=[
            pltpu.VMEM((_CH,), jnp.int32),
            pltpu.VMEM((_CH,), jnp.int32),
            pltpu.VMEM((_CH, H), jnp.int32),
            pltpu.VMEM((_CH, H), jnp.int32),
            pltpu.SemaphoreType.DMA,
            pltpu.SemaphoreType.DMA,
        ],
    )
    return f(ps, pd, src, dst)


def _unpack(p):
    lo = lax.bitcast_convert_type(lax.shift_left(p, 16), jnp.float32)
    hi = lax.bitcast_convert_type(
        lax.bitwise_and(p, jnp.int32(-65536)), jnp.float32)
    return lo, hi


def _stage_c_body(ea, eb, wce, wco, b2cat, g):
    alo, ahi = _unpack(ea[...])
    blo, bhi = _unpack(eb[...])
    ze = jax.nn.relu(alo + blo)
    zo = jax.nn.relu(ahi + bhi)
    h = ze @ wce[...] + zo @ wco[...] + b2cat[...]
    k = jax.nn.sigmoid(h[:, 3 * H:3 * H + 1])
    g[...] = h[:, 0:3 * H] * k


def _stage_c(ea, eb, w2cat, b2cat):
    blk = 512
    grid = E_TOT // blk
    espec = pl.BlockSpec((blk, H), lambda i: (i, 0))
    return pl.pallas_call(
        _stage_c_body,
        grid=(grid,),
        in_specs=[espec, espec,
                  pl.BlockSpec((H, 4 * H), lambda i: (0, 0)),
                  pl.BlockSpec((H, 4 * H), lambda i: (0, 0)),
                  pl.BlockSpec((1, 4 * H), lambda i: (0, 0))],
        out_specs=pl.BlockSpec((blk, 3 * H), lambda i: (i, 0)),
        out_shape=jax.ShapeDtypeStruct((E_TOT, 3 * H), jnp.float32),
    )(ea, eb, w2cat[0::2], w2cat[1::2], b2cat)


def _stage_e_body(gcx, msc, a2c, a3c, gsx, mss, a2s, a3s,
                  rwc0, rwc2, rwc3, rbc, rws0, rws2, rws3, rbs,
                  cw1a, cw1b, cb1, cw2, cb2, sw1a, sw1b, sb1, sw2, sb2,
                  out_fc, out_fs):
    new_cx = gcx[...] @ rwc0[...] + msc[...] + a2c[...] @ rwc2[...] \
        + a3c[...] @ rwc3[...] + rbc[...]
    t = jax.nn.relu(gcx[...] @ cw1a[...] + new_cx @ cw1b[...] + cb1[...])
    out_fc[...] = t @ cw2[...] + cb2[...]
    new_sx = gsx[...] @ rws0[...] + mss[...] + a2s[...] @ rws2[...] \
        + a3s[...] @ rws3[...] + rbs[...]
    u = jax.nn.relu(gsx[...] @ sw1a[...] + new_sx @ sw1b[...] + sb1[...])
    out_fs[...] = u @ sw2[...] + sb2[...]


def _stage_e(gc_x, msc, a2c, a3c, gs_x, mss, a2s, a3s,
             red_s2c_w, red_s2c_b, red_c2s_w, red_c2s_b,
             gc_w1, gc_b1, gc_w2, gc_b2, gs_w1, gs_b1, gs_w2, gs_b2):
    n = gc_x.shape[0]
    blk = 1000
    grid = n // blk
    row = pl.BlockSpec((blk, H), lambda i: (i, 0))
    wfull = pl.BlockSpec((H, H), lambda i: (0, 0))
    bfull = pl.BlockSpec((1, H), lambda i: (0, 0))
    return pl.pallas_call(
        _stage_e_body,
        grid=(grid,),
        in_specs=[row] * 8 + [wfull, wfull, wfull, bfull] * 2
        + [wfull, wfull, bfull, wfull, bfull] * 2,
        out_specs=(row, row),
        out_shape=(jax.ShapeDtypeStruct((n, H), jnp.float32),
                   jax.ShapeDtypeStruct((n, H), jnp.float32)),
    )(gc_x, msc, a2c, a3c, gs_x, mss, a2s, a3s,
      red_s2c_w[0:H], red_s2c_w[2 * H:3 * H], red_s2c_w[3 * H:4 * H],
      red_s2c_b.reshape(1, H),
      red_c2s_w[0:H], red_c2s_w[2 * H:3 * H], red_c2s_w[3 * H:4 * H],
      red_c2s_b.reshape(1, H),
      gc_w1[0:H], gc_w1[H:2 * H], gc_b1.reshape(1, H), gc_w2,
      gc_b2.reshape(1, H),
      gs_w1[0:H], gs_w1[H:2 * H], gs_b1.reshape(1, H), gs_w2,
      gs_b2.reshape(1, H))


def _edge_payload(ps, pd, src, dst, w2cat, b2cat):
    ea, eb = _sc_gather(ps, pd, src, dst)
    return _stage_c(ea, eb, w2cat, b2cat)


def kernel(nf_gc, nf_gs, ei_s2c, ei_c2s, lin_gc_w, lin_gc_b, lin_gs_w, lin_gs_b,
           msg_s2c_w1, msg_s2c_b1, msg_s2c_w2, msg_s2c_b2,
           red_s2c_w, red_s2c_b,
           msg_c2s_w1, msg_c2s_b1, msg_c2s_w2, msg_c2s_b2,
           red_c2s_w, red_c2s_b,
           gc_w1, gc_b1, gc_w2, gc_b2,
           gs_w1, gs_b1, gs_w2, gs_b2):
    gc_x, gs_x, ps_s2c, pd_s2c, ps_c2s, pd_c2s = _stage_a(
        nf_gc, nf_gs, lin_gc_w, lin_gc_b, lin_gs_w, lin_gs_b,
        msg_s2c_w1, msg_s2c_b1, msg_c2s_w1, msg_c2s_b1)
    w2cat_c, b2cat_c = _prep(msg_s2c_w2, msg_s2c_b2, red_s2c_w)
    w2cat_s, b2cat_s = _prep(msg_c2s_w2, msg_c2s_b2, red_c2s_w)

    pack = lambda t: jax.lax.bitcast_convert_type(
        t.reshape(N_NODE, H, 2), jnp.int32)
    g_c = _edge_payload(pack(ps_s2c), pack(pd_s2c),
                        ei_s2c[0], ei_s2c[1], w2cat_c, b2cat_c)
    g_s = _edge_payload(pack(ps_c2s), pack(pd_c2s),
                        ei_c2s[0], ei_c2s[1], w2cat_s, b2cat_s)

    dst_c = ei_s2c[1]
    dst_s = ei_c2s[1]
    msc = jax.ops.segment_sum(g_c[:, 0:H], dst_c, num_segments=N_NODE)
    a2c = jax.ops.segment_max(g_c[:, H:2 * H], dst_c, num_segments=N_NODE)
    a3c = jax.ops.segment_min(g_c[:, 2 * H:3 * H], dst_c, num_segments=N_NODE)
    mss = jax.ops.segment_sum(g_s[:, 0:H], dst_s, num_segments=N_NODE)
    a2s = jax.ops.segment_max(g_s[:, H:2 * H], dst_s, num_segments=N_NODE)
    a3s = jax.ops.segment_min(g_s[:, 2 * H:3 * H], dst_s, num_segments=N_NODE)

    return _stage_e(gc_x, msc, a2c, a3c, gs_x, mss, a2s, a3s,
                    red_s2c_w, red_s2c_b, red_c2s_w, red_c2s_b,
                    gc_w1, gc_b1, gc_w2, gc_b2, gs_w1, gs_b1, gs_w2, gs_b2)
